# baseline (speedup 1.0000x reference)
import functools

import jax
import jax.numpy as jnp
from jax import lax
from jax.experimental import pallas as pl
from jax.experimental.pallas import tpu as pltpu
from jax.experimental.pallas import tpu_sc as plsc

N = 10000
E = 320000
D = 128
C = 64

NC = 2
NS = 16
NW = NC * NS
CHUNK = 128
EPAD = ((E + NW * CHUNK * 8 - 1) // (NW * CHUNK * 8)) * (NW * CHUNK * 8)
NCHUNKS = EPAD // CHUNK
CPW = NCHUNKS // NW
NPAD = 10112
RPT = NPAD // NS


def _sc_prop_body(table, src_idx, dst_idx, zeros, out, src_slab, dst_slab,
                  rows, gsem, acc):
    c = lax.axis_index("c")
    s = lax.axis_index("s")
    w = s * NC + c
    r0 = s * RPT
    pltpu.sync_copy(zeros, acc.at[pl.ds(r0, RPT)])
    pltpu.sync_copy(src_idx.at[pl.ds(w * CPW, CPW)], src_slab)
    pltpu.sync_copy(dst_idx.at[pl.ds(w * CPW, CPW)], dst_slab)
    plsc.subcore_barrier()

    def chunk(i, carry):
        pltpu.async_copy(table.at[src_slab.at[i]], rows, gsem).wait()
        pltpu.sync_copy(rows, acc.at[dst_slab.at[i]], add=True)
        return carry

    lax.fori_loop(0, CPW, chunk, 0)
    plsc.subcore_barrier()
    pltpu.sync_copy(acc.at[pl.ds(r0, RPT)], out.at[c, pl.ds(r0, RPT)])


_sc_prop = functools.partial(
    pl.kernel,
    mesh=plsc.VectorSubcoreMesh(core_axis_name="c", subcore_axis_name="s"),
    out_type=jax.ShapeDtypeStruct((NC, NPAD, C), jnp.float32),
    scratch_types=[
        pltpu.VMEM((CPW, CHUNK), jnp.int32),
        pltpu.VMEM((CPW, CHUNK), jnp.int32),
        pltpu.VMEM((CHUNK, C), jnp.float32),
        pltpu.SemaphoreType.DMA,
        pltpu.VMEM_SHARED((NPAD, C), jnp.float32),
    ],
    compiler_params=pltpu.CompilerParams(use_tc_tiling_on_sc=False),
)(_sc_prop_body)


def _mm_body(x_ref, wt_ref, o_ref):
    o_ref[...] = jnp.dot(x_ref[...], wt_ref[...],
                         preferred_element_type=jnp.float32)


def _linear(feat, wt):
    return pl.pallas_call(
        _mm_body,
        out_shape=jax.ShapeDtypeStruct((N, C), jnp.float32),
    )(feat, wt)


def _comb_body(p_ref, b_ref, o_ref):
    o_ref[...] = p_ref[0, :N, :] + p_ref[1, :N, :] + b_ref[...]


def _combine(partials, bias2d):
    return pl.pallas_call(
        _comb_body,
        out_shape=jax.ShapeDtypeStruct((N, C), jnp.float32),
    )(partials, bias2d)


def kernel(feat, edge_index, feat_ori, W, b):
    src = edge_index[0]
    dst = edge_index[1]
    src_p = jnp.concatenate(
        [src, jnp.zeros((EPAD - E,), jnp.int32)]).reshape(NCHUNKS, CHUNK)
    pad_dst = N + jnp.arange(EPAD - E, dtype=jnp.int32) % (NPAD - N)
    dst_p = jnp.concatenate([dst, pad_dst]).reshape(NCHUNKS, CHUNK)
    zeros = jnp.zeros((RPT, C), jnp.float32)

    y0 = _linear(feat, W.T)
    p1 = _sc_prop(y0, src_p, dst_p, zeros)
    h1 = _combine(p1, jnp.zeros((1, C), jnp.float32))
    p2 = _sc_prop(h1, src_p, dst_p, zeros)
    out = _combine(p2, b.reshape(1, C))
    return out


# speedup vs baseline: 1.1651x; 1.1651x over previous
import functools

import jax
import jax.numpy as jnp
from jax import lax
from jax.experimental import pallas as pl
from jax.experimental.pallas import tpu as pltpu
from jax.experimental.pallas import tpu_sc as plsc

N = 10000
E = 320000
D = 128
C = 64

NC = 2
NS = 16
NW = NC * NS
CHUNK = 128
EPAD = ((E + NW * CHUNK * 8 - 1) // (NW * CHUNK * 8)) * (NW * CHUNK * 8)
NCHUNKS = EPAD // CHUNK
CPW = NCHUNKS // NW
NPAD = 10112
RPT = NPAD // NS


def _sc_prop_body(table, src_idx, dst_idx, zeros, out, src_slab, dst_slab,
                  rows0, rows1, sem0, sem1, acc):
    c = lax.axis_index("c")
    s = lax.axis_index("s")
    w = s * NC + c
    r0 = s * RPT
    pltpu.sync_copy(zeros, acc.at[pl.ds(r0, RPT)])
    pltpu.sync_copy(src_idx.at[pl.ds(w * CPW, CPW)], src_slab)
    pltpu.sync_copy(dst_idx.at[pl.ds(w * CPW, CPW)], dst_slab)
    plsc.subcore_barrier()

    pltpu.async_copy(table.at[src_slab.at[0]], rows0, sem0)

    def body2(j, carry):
        i = 2 * j
        i1 = i + 1
        i2 = jnp.minimum(i + 2, CPW - 1)
        pltpu.async_copy(table.at[src_slab.at[i1]], rows1, sem1)
        pltpu.make_async_copy(table.at[src_slab.at[i]], rows0, sem0).wait()
        pltpu.sync_copy(rows0, acc.at[dst_slab.at[i]], add=True)
        pltpu.async_copy(table.at[src_slab.at[i2]], rows0, sem0)
        pltpu.make_async_copy(table.at[src_slab.at[i1]], rows1, sem1).wait()
        pltpu.sync_copy(rows1, acc.at[dst_slab.at[i1]], add=True)
        return carry

    lax.fori_loop(0, CPW // 2, body2, 0)
    pltpu.make_async_copy(table.at[src_slab.at[CPW - 1]], rows0, sem0).wait()
    plsc.subcore_barrier()
    pltpu.sync_copy(acc.at[pl.ds(r0, RPT)], out.at[c, pl.ds(r0, RPT)])


_sc_prop = functools.partial(
    pl.kernel,
    mesh=plsc.VectorSubcoreMesh(core_axis_name="c", subcore_axis_name="s"),
    out_type=jax.ShapeDtypeStruct((NC, NPAD, C), jnp.float32),
    scratch_types=[
        pltpu.VMEM((CPW, CHUNK), jnp.int32),
        pltpu.VMEM((CPW, CHUNK), jnp.int32),
        pltpu.VMEM((CHUNK, C), jnp.float32),
        pltpu.VMEM((CHUNK, C), jnp.float32),
        pltpu.SemaphoreType.DMA,
        pltpu.SemaphoreType.DMA,
        pltpu.VMEM_SHARED((NPAD, C), jnp.float32),
    ],
    compiler_params=pltpu.CompilerParams(use_tc_tiling_on_sc=False),
)(_sc_prop_body)


def _mm_body(x_ref, wt_ref, o_ref):
    o_ref[...] = jnp.dot(x_ref[...], wt_ref[...],
                         preferred_element_type=jnp.float32)


def _linear(feat, wt):
    return pl.pallas_call(
        _mm_body,
        out_shape=jax.ShapeDtypeStruct((N, C), jnp.float32),
    )(feat, wt)


def _comb_body(p_ref, b_ref, o_ref):
    o_ref[...] = p_ref[0, :N, :] + p_ref[1, :N, :] + b_ref[...]


def _combine(partials, bias2d):
    return pl.pallas_call(
        _comb_body,
        out_shape=jax.ShapeDtypeStruct((N, C), jnp.float32),
    )(partials, bias2d)


def kernel(feat, edge_index, feat_ori, W, b):
    src = edge_index[0]
    dst = edge_index[1]
    src_p = jnp.concatenate(
        [src, jnp.zeros((EPAD - E,), jnp.int32)]).reshape(NCHUNKS, CHUNK)
    pad_dst = N + jnp.arange(EPAD - E, dtype=jnp.int32) % (NPAD - N)
    dst_p = jnp.concatenate([dst, pad_dst]).reshape(NCHUNKS, CHUNK)
    zeros = jnp.zeros((RPT, C), jnp.float32)

    y0 = _linear(feat, W.T)
    p1 = _sc_prop(y0, src_p, dst_p, zeros)
    h1 = _combine(p1, jnp.zeros((1, C), jnp.float32))
    p2 = _sc_prop(h1, src_p, dst_p, zeros)
    out = _combine(p2, b.reshape(1, C))
    return out


# speedup vs baseline: 1.2543x; 1.0765x over previous
import functools

import jax
import jax.numpy as jnp
from jax import lax
from jax.experimental import pallas as pl
from jax.experimental.pallas import tpu as pltpu
from jax.experimental.pallas import tpu_sc as plsc

N = 10000
E = 320000
D = 128
C = 64

NC = 2
NS = 16
NW = NC * NS
CHUNK = 128
EPAD = ((E + NW * CHUNK * 8 - 1) // (NW * CHUNK * 8)) * (NW * CHUNK * 8)
NCHUNKS = EPAD // CHUNK
CPW0 = 120
CPW1 = NCHUNKS // NS - CPW0
BASE1 = NS * CPW0
NPAD = 10112
RPT = NPAD // NS


def _sc_prop_body(table, src_idx, dst_idx, zeros, out, src_slab, dst_slab,
                  rows0, rows1, sem0, sem1, acc):
    c = lax.axis_index("c")
    s = lax.axis_index("s")
    r0 = s * RPT
    pltpu.sync_copy(zeros, acc.at[pl.ds(r0, RPT)])
    plsc.subcore_barrier()

    def run(cpw, base):
        pltpu.sync_copy(src_idx.at[pl.ds(base, cpw)], src_slab.at[pl.ds(0, cpw)])
        pltpu.sync_copy(dst_idx.at[pl.ds(base, cpw)], dst_slab.at[pl.ds(0, cpw)])
        pltpu.async_copy(table.at[src_slab.at[0]], rows0, sem0)

        def body2(j, carry):
            i = 2 * j
            i1 = i + 1
            i2 = jnp.minimum(i + 2, cpw - 1)
            pltpu.async_copy(table.at[src_slab.at[i1]], rows1, sem1)
            pltpu.make_async_copy(table.at[src_slab.at[i]], rows0, sem0).wait()
            pltpu.sync_copy(rows0, acc.at[dst_slab.at[i]], add=True)
            pltpu.async_copy(table.at[src_slab.at[i2]], rows0, sem0)
            pltpu.make_async_copy(table.at[src_slab.at[i1]], rows1, sem1).wait()
            pltpu.sync_copy(rows1, acc.at[dst_slab.at[i1]], add=True)
            return carry

        lax.fori_loop(0, cpw // 2, body2, 0)
        pltpu.make_async_copy(table.at[src_slab.at[cpw - 1]], rows0, sem0).wait()

    @pl.when(c == 0)
    def _():
        run(CPW0, s * CPW0)

    @pl.when(c == 1)
    def _():
        run(CPW1, BASE1 + s * CPW1)
    plsc.subcore_barrier()
    pltpu.sync_copy(acc.at[pl.ds(r0, RPT)], out.at[c, pl.ds(r0, RPT)])


_sc_prop = functools.partial(
    pl.kernel,
    mesh=plsc.VectorSubcoreMesh(core_axis_name="c", subcore_axis_name="s"),
    out_type=jax.ShapeDtypeStruct((NC, NPAD, C), jnp.float32),
    scratch_types=[
        pltpu.VMEM((CPW0, CHUNK), jnp.int32),
        pltpu.VMEM((CPW0, CHUNK), jnp.int32),
        pltpu.VMEM((CHUNK, C), jnp.float32),
        pltpu.VMEM((CHUNK, C), jnp.float32),
        pltpu.SemaphoreType.DMA,
        pltpu.SemaphoreType.DMA,
        pltpu.VMEM_SHARED((NPAD, C), jnp.float32),
    ],
    compiler_params=pltpu.CompilerParams(use_tc_tiling_on_sc=False),
)(_sc_prop_body)


def _mm_body(x_ref, wt_ref, o_ref):
    o_ref[...] = jnp.dot(x_ref[...], wt_ref[...],
                         preferred_element_type=jnp.float32)


def _linear(feat, wt):
    return pl.pallas_call(
        _mm_body,
        out_shape=jax.ShapeDtypeStruct((N, C), jnp.float32),
    )(feat, wt)


def _comb_body(p_ref, b_ref, o_ref):
    o_ref[...] = p_ref[0, :N, :] + p_ref[1, :N, :] + b_ref[...]


def _combine(partials, bias2d):
    return pl.pallas_call(
        _comb_body,
        out_shape=jax.ShapeDtypeStruct((N, C), jnp.float32),
    )(partials, bias2d)


def kernel(feat, edge_index, feat_ori, W, b):
    src = edge_index[0]
    dst = edge_index[1]
    src_p = jnp.concatenate(
        [src, jnp.zeros((EPAD - E,), jnp.int32)]).reshape(NCHUNKS, CHUNK)
    pad_dst = N + jnp.arange(EPAD - E, dtype=jnp.int32) % (NPAD - N)
    dst_p = jnp.concatenate([dst, pad_dst]).reshape(NCHUNKS, CHUNK)
    zeros = jnp.zeros((RPT, C), jnp.float32)

    y0 = _linear(feat, W.T)
    p1 = _sc_prop(y0, src_p, dst_p, zeros)
    h1 = _combine(p1, jnp.zeros((1, C), jnp.float32))
    p2 = _sc_prop(h1, src_p, dst_p, zeros)
    out = _combine(p2, b.reshape(1, C))
    return out
